# baseline (speedup 1.0000x reference)
import functools

import jax
import jax.numpy as jnp
from jax import lax
from jax.experimental import pallas as pl
from jax.experimental.pallas import tpu as pltpu
from jax.experimental.pallas import tpu_sc as plsc

_EMB = 64
_BATCH = 4096
_HIST = 50
_NTOT = _BATCH * _HIST

_info = plsc.get_sparse_core_info()
_NC, _NS = _info.num_cores, _info.num_subcores
_NW = _NC * _NS
_B_PER_W = _NTOT // _NW
_CHUNK = 800
_NCHUNK = _B_PER_W // _CHUNK

_mesh = plsc.VectorSubcoreMesh(core_axis_name="c", subcore_axis_name="s")


@functools.partial(
    pl.kernel,
    mesh=_mesh,
    out_type=jax.ShapeDtypeStruct((_NTOT,), jnp.int32),
    scratch_types=[
        pltpu.VMEM((8, 128), jnp.int32),
    ],
)
def _detile_idx(idxt_hbm, out_hbm, tile_v):
    wid = lax.axis_index("s") * _NC + lax.axis_index("c")
    col = wid * 128
    for a in range(7):
        rows = 8 if a < 6 else 2
        pltpu.sync_copy(
            idxt_hbm.at[pl.ds(a * 8, rows), pl.ds(col, 128)],
            tile_v.at[pl.ds(0, rows)],
        )
        for s in range(rows):
            pltpu.sync_copy(
                tile_v.at[s],
                out_hbm.at[pl.ds((a * 8 + s) * _BATCH + col, 128)],
            )


@functools.partial(
    pl.kernel,
    mesh=_mesh,
    out_type=jax.ShapeDtypeStruct((_NTOT, _EMB), jnp.float32),
    scratch_types=[
        pltpu.VMEM((_B_PER_W,), jnp.int32),
        pltpu.VMEM((2, _CHUNK, _EMB), jnp.float32),
        pltpu.SemaphoreType.DMA,
        pltpu.SemaphoreType.DMA,
    ],
    compiler_params=pltpu.CompilerParams(use_tc_tiling_on_sc=False),
)
def _emb_lookup(idx_hbm, table_hbm, out_hbm, idx_v, rows_v, gsem0, gsem1):
    wid = lax.axis_index("s") * _NC + lax.axis_index("c")
    base = wid * _B_PER_W
    pltpu.sync_copy(idx_hbm.at[pl.ds(base, _B_PER_W)], idx_v)

    pltpu.async_copy(
        table_hbm.at[idx_v.at[pl.ds(0, _CHUNK)]], rows_v.at[0], gsem0
    )

    def body(p, _):
        c0 = 2 * p
        pltpu.async_copy(
            table_hbm.at[idx_v.at[pl.ds((c0 + 1) * _CHUNK, _CHUNK)]],
            rows_v.at[1],
            gsem1,
        )
        pltpu.make_async_copy(
            table_hbm.at[idx_v.at[pl.ds(0, _CHUNK)]], rows_v.at[0], gsem0
        ).wait()
        pltpu.sync_copy(
            rows_v.at[0], out_hbm.at[pl.ds(base + c0 * _CHUNK, _CHUNK)]
        )
        nxt = lax.min(c0 + 2, _NCHUNK - 2)
        pltpu.async_copy(
            table_hbm.at[idx_v.at[pl.ds(nxt * _CHUNK, _CHUNK)]],
            rows_v.at[0],
            gsem0,
        )
        pltpu.make_async_copy(
            table_hbm.at[idx_v.at[pl.ds(0, _CHUNK)]], rows_v.at[1], gsem1
        ).wait()
        pltpu.sync_copy(
            rows_v.at[1], out_hbm.at[pl.ds(base + (c0 + 1) * _CHUNK, _CHUNK)]
        )
        return _

    lax.fori_loop(0, _NCHUNK // 2, body, 0)
    pltpu.make_async_copy(
        table_hbm.at[idx_v.at[pl.ds(0, _CHUNK)]], rows_v.at[0], gsem0
    ).wait()


def kernel(input, weight):
    idxt = jnp.transpose(input.astype(jnp.int32))
    flat = _detile_idx(idxt)
    out = _emb_lookup(flat, weight)
    out3 = jnp.reshape(out, (_HIST, _BATCH, _EMB))
    return jnp.transpose(out3, (1, 0, 2))
